# baseline (speedup 1.0000x reference)
import functools

import jax
import jax.numpy as jnp
from jax import lax
from jax.experimental import pallas as pl
from jax.experimental.pallas import tpu as pltpu
from jax.experimental.pallas import tpu_sc as plsc

_RADIUS = 0.2
_K = 32
_EPS = 1e-5

_B, _N, _M = 2, 8192, 2048
_W = 32
_NW = 32
_MPW = _M // _NW
_ROWS = _B * _M * _K
_RT = 8192
_GRID = _ROWS // _RT
_HIGHEST = lax.Precision.HIGHEST
_DEFAULT = lax.Precision.DEFAULT


def _make_ball_query():
  mesh = plsc.VectorSubcoreMesh(core_axis_name="c", subcore_axis_name="s")

  @functools.partial(
      pl.kernel,
      mesh=mesh,
      compiler_params=pltpu.CompilerParams(needs_layout_passes=False),
      out_type=(
          jax.ShapeDtypeStruct((_B * _M * _K,), jnp.int32),
          jax.ShapeDtypeStruct((_B * _M,), jnp.int32),
      ),
      scratch_types=[
          pltpu.VMEM((_N,), jnp.float32),
          pltpu.VMEM((_N,), jnp.float32),
          pltpu.VMEM((_N,), jnp.float32),
          pltpu.VMEM((_MPW,), jnp.float32),
          pltpu.VMEM((_MPW,), jnp.float32),
          pltpu.VMEM((_MPW,), jnp.float32),
          pltpu.VMEM((128,), jnp.int32),
          pltpu.VMEM((_MPW * _K,), jnp.int32),
          pltpu.VMEM((_MPW,), jnp.int32),
      ],
  )
  def ball_query_kernel(xyzf_hbm, newf_hbm, idx_hbm, cnt_hbm,
                        x_v, y_v, z_v, cx_v, cy_v, cz_v,
                        list_v, oidx_v, ocnt_v):
    wid = lax.axis_index("s") * 2 + lax.axis_index("c")
    iota = lax.iota(jnp.int32, 16)
    zeros16 = jnp.zeros((16,), jnp.int32)
    r2 = jnp.float32(_RADIUS * _RADIUS)
    for b in range(_B):
      pltpu.sync_copy(xyzf_hbm.at[pl.ds((b * 3 + 0) * _N, _N)], x_v)
      pltpu.sync_copy(xyzf_hbm.at[pl.ds((b * 3 + 1) * _N, _N)], y_v)
      pltpu.sync_copy(xyzf_hbm.at[pl.ds((b * 3 + 2) * _N, _N)], z_v)
      m0 = wid * _MPW
      pltpu.sync_copy(newf_hbm.at[pl.ds((b * 3 + 0) * _M + m0, _MPW)], cx_v)
      pltpu.sync_copy(newf_hbm.at[pl.ds((b * 3 + 1) * _M + m0, _MPW)], cy_v)
      pltpu.sync_copy(newf_hbm.at[pl.ds((b * 3 + 2) * _M + m0, _MPW)], cz_v)

      def per_centroid(i, carry):
        list_v[pl.ds(0, 16)] = zeros16
        ci = lax.broadcast(i, (16,))
        cx = plsc.load_gather(cx_v, [ci])
        cy = plsc.load_gather(cy_v, [ci])
        cz = plsc.load_gather(cz_v, [ci])

        def cond(c):
          n, cs, _ = c
          return jnp.logical_and(n < _N, cs < _K)

        def body(c):
          n, cs, cv = c
          for u in range(4):
            nv = lax.broadcast(n + u * 16, (16,)) + iota
            xv = plsc.load_gather(x_v, [nv])
            yv = plsc.load_gather(y_v, [nv])
            zv = plsc.load_gather(z_v, [nv])
            dx = xv - cx
            dy = yv - cy
            dz = zv - cz
            d2 = dx * dx + dy * dy + dz * dz
            msk = d2 < r2
            pos = cv + plsc.cumsum(msk.astype(jnp.int32)) - 1
            plsc.store_scatter(list_v, [pos], nv, mask=msk)
            cv = cv + plsc.all_reduce_population_count(msk)
          return (n + 64, jnp.max(cv), cv)

        _, cs, _ = lax.while_loop(
            cond, body, (jnp.int32(0), jnp.int32(0), zeros16))
        cnt = jnp.minimum(cs, _K)
        cntv = lax.broadcast(cnt, (16,))
        first = plsc.load_gather(list_v, [zeros16])
        for h in range(2):
          slots = iota + (h * 16)
          vals = list_v[pl.ds(h * 16, 16)]
          vals = jnp.where(slots < cntv, vals, first) + (b * _N)
          plsc.store_scatter(
              oidx_v, [lax.broadcast(i * _K + h * 16, (16,)) + iota], vals)
        plsc.store_scatter(ocnt_v, [ci], cntv, mask=(iota == 0))
        return carry

      lax.fori_loop(0, _MPW, per_centroid, 0)
      g0 = b * _M + wid * _MPW
      pltpu.sync_copy(oidx_v, idx_hbm.at[pl.ds(g0 * _K, _MPW * _K)])
      pltpu.sync_copy(ocnt_v, cnt_hbm.at[pl.ds(g0, _MPW)])

  return ball_query_kernel


def _make_gather():
  mesh = plsc.VectorSubcoreMesh(core_axis_name="c", subcore_axis_name="s")
  ch = 128
  rpw = _ROWS // _NW
  nch = rpw // ch

  @functools.partial(
      pl.kernel,
      mesh=mesh,
      compiler_params=pltpu.CompilerParams(
          needs_layout_passes=False, use_tc_tiling_on_sc=False),
      out_type=jax.ShapeDtypeStruct((_ROWS, _W), jnp.float32),
      scratch_types=[
          pltpu.VMEM((ch,), jnp.int32),
          pltpu.VMEM((ch, _W), jnp.float32),
          pltpu.SemaphoreType.DMA,
      ],
  )
  def gather_kernel(tab_hbm, idx_hbm, out_hbm, idx_v, rows_v, sem):
    wid = lax.axis_index("s") * 2 + lax.axis_index("c")
    r0 = wid * rpw

    def chunk(c, carry):
      base = r0 + c * ch
      pltpu.sync_copy(idx_hbm.at[pl.ds(base, ch)], idx_v)
      pltpu.async_copy(tab_hbm.at[idx_v], rows_v, sem).wait()
      pltpu.sync_copy(rows_v, out_hbm.at[pl.ds(base, ch)])
      return carry

    lax.fori_loop(0, nch, chunk, 0)

  return gather_kernel


_P = 8
_PW = _P * _W
_PROWS = _ROWS // _P
_CHP = 8192
_NCH = _PROWS // _CHP
_MTC = _CHP * _P // _K


def _true_stats(gp, sp, width=_W):
  g = jnp.zeros((width, width), jnp.float32)
  s = jnp.zeros((1, width), jnp.float32)
  for d in range(_P):
    g = g + lax.slice(gp, (d * width, d * width),
                      ((d + 1) * width, (d + 1) * width))
    s = s + lax.slice(sp, (0, d * width), (1, (d + 1) * width))
  return g, s


def _bn_affine(g_mat, s_vec, w_mat, gamma, beta):
  mean = jnp.dot(s_vec, w_mat, precision=_HIGHEST) / _ROWS
  m2 = jnp.sum(w_mat * jnp.dot(g_mat, w_mat, precision=_HIGHEST),
               axis=0, keepdims=True) / _ROWS
  var = m2 - mean * mean
  scale = gamma * lax.rsqrt(var + _EPS)
  shift = beta - mean * scale
  return scale, shift


def _bn_packed(gp_ref, sp_ref, w_ref, gamma_ref, beta_ref, wpad=None):
  g, s = _true_stats(gp_ref[...], sp_ref[...])
  w = w_ref[...] if wpad is None else w_ref[...][:_W - wpad, :]
  sc, sh = _bn_affine(g, s, w, gamma_ref[...], beta_ref[...])
  return (jnp.concatenate([sc] * _P, axis=1),
          jnp.concatenate([sh] * _P, axis=1))


def _xcp(xr_ref, c_ref):
  x = xr_ref[...]
  cw = c_ref[...]
  cexp = jnp.broadcast_to(
      cw[:, None, :], (_MTC, _K // _P, _PW)).reshape(_CHP, _PW)
  return x - cexp


def _accum(i, g_ref, s_ref, h):
  @pl.when(i == 0)
  def _():
    g_ref[...] = jnp.zeros_like(g_ref)
    s_ref[...] = jnp.zeros_like(s_ref)
  g_ref[...] += lax.dot_general(h, h, (((0,), (0,)), ((), ())),
                                precision=_DEFAULT)
  s_ref[...] += jnp.sum(h, axis=0, keepdims=True)


def _full(shape):
  return pl.BlockSpec(shape, lambda i: tuple(0 for _ in shape))


_XRP_SPEC = pl.BlockSpec((_CHP, _PW), lambda i: (i, 0))
_CP_SPEC = pl.BlockSpec((_MTC, _PW), lambda i: (i, 0))
_GP_OUT = [_full((_PW, _PW)), _full((1, _PW))]
_GP_SHAPE = [jax.ShapeDtypeStruct((_PW, _PW), jnp.float32),
             jax.ShapeDtypeStruct((1, _PW), jnp.float32)]


def _p1(xrp, cp):
  def body(xr_ref, c_ref, g_ref, s_ref):
    _accum(pl.program_id(0), g_ref, s_ref, _xcp(xr_ref, c_ref))

  return pl.pallas_call(
      body, grid=(_NCH,),
      in_specs=[_XRP_SPEC, _CP_SPEC],
      out_specs=_GP_OUT, out_shape=_GP_SHAPE,
  )(xrp, cp)


def _p2(xrp, cp, wd1, w1, g1, b1, gx, sx):
  def body(xr_ref, c_ref, wd1_ref, w1_ref, g1_ref, b1_ref, gx_ref, sx_ref,
           g_ref, s_ref, bn_s):
    i = pl.program_id(0)

    @pl.when(i == 0)
    def _():
      sc, sh = _bn_packed(gx_ref, sx_ref, w1_ref, g1_ref, b1_ref)
      bn_s[0:1, :] = sc
      bn_s[1:2, :] = sh

    h1 = jnp.maximum(
        jnp.dot(_xcp(xr_ref, c_ref), wd1_ref[...],
                precision=_DEFAULT) * bn_s[0:1, :] + bn_s[1:2, :], 0.0)
    _accum(i, g_ref, s_ref, h1)

  return pl.pallas_call(
      body, grid=(_NCH,),
      in_specs=[_XRP_SPEC, _CP_SPEC, _full((_PW, _PW)), _full((_W, 32)),
                _full((1, 32)), _full((1, 32)), _full((_PW, _PW)),
                _full((1, _PW))],
      out_specs=_GP_OUT, out_shape=_GP_SHAPE,
      scratch_shapes=[pltpu.VMEM((2, _PW), jnp.float32)],
  )(xrp, cp, wd1, w1, g1, b1, gx, sx)


def _p3(xrp, cp, wd1, w1, g1, b1, wd2, w2, g2, b2, gx, sx, gh1, sh1):
  def body(xr_ref, c_ref, wd1_ref, w1_ref, g1_ref, b1_ref,
           wd2_ref, w2_ref, g2_ref, b2_ref,
           gx_ref, sx_ref, gh1_ref, sh1_ref, g_ref, s_ref, bn_s):
    i = pl.program_id(0)

    @pl.when(i == 0)
    def _():
      sc1, sh1_ = _bn_packed(gx_ref, sx_ref, w1_ref, g1_ref, b1_ref)
      sc2, sh2_ = _bn_packed(gh1_ref, sh1_ref, w2_ref, g2_ref, b2_ref)
      bn_s[0:1, :] = sc1
      bn_s[1:2, :] = sh1_
      bn_s[2:3, :] = sc2
      bn_s[3:4, :] = sh2_

    h1 = jnp.maximum(
        jnp.dot(_xcp(xr_ref, c_ref), wd1_ref[...],
                precision=_DEFAULT) * bn_s[0:1, :] + bn_s[1:2, :], 0.0)
    h2 = jnp.maximum(
        jnp.dot(h1, wd2_ref[...],
                precision=_DEFAULT) * bn_s[2:3, :] + bn_s[3:4, :], 0.0)
    _accum(i, g_ref, s_ref, h2)

  return pl.pallas_call(
      body, grid=(_NCH,),
      in_specs=[_XRP_SPEC, _CP_SPEC, _full((_PW, _PW)), _full((_W, 32)),
                _full((1, 32)), _full((1, 32)), _full((_PW, _PW)),
                _full((32, 32)), _full((1, 32)), _full((1, 32)),
                _full((_PW, _PW)), _full((1, _PW)),
                _full((_PW, _PW)), _full((1, _PW))],
      out_specs=_GP_OUT, out_shape=_GP_SHAPE,
      scratch_shapes=[pltpu.VMEM((4, _PW), jnp.float32)],
  )(xrp, cp, wd1, w1, g1, b1, wd2, w2, g2, b2, gx, sx, gh1, sh1)


def _p4(xrp, cp, wd1, w1, g1, b1, wd2, w2, g2, b2, wd3, w3, g3, b3,
        gx, sx, gh1, sh1, gh2, sh2, cnt2):
  pw3 = _P * 64

  def body(xr_ref, c_ref, wd1_ref, w1_ref, g1_ref, b1_ref,
           wd2_ref, w2_ref, g2_ref, b2_ref, wd3_ref, w3_ref, g3_ref, b3_ref,
           gx_ref, sx_ref, gh1_ref, sh1_ref, gh2_ref, sh2_ref,
           cnt_ref, o_ref, bn_s, bn3_s):
    i = pl.program_id(0)

    @pl.when(i == 0)
    def _():
      sc1, sh1_ = _bn_packed(gx_ref, sx_ref, w1_ref, g1_ref, b1_ref)
      sc2, sh2_ = _bn_packed(gh1_ref, sh1_ref, w2_ref, g2_ref, b2_ref)
      g3t, s3t = _true_stats(gh2_ref[...], sh2_ref[...])
      sc3, sh3_ = _bn_affine(g3t, s3t, w3_ref[...], g3_ref[...], b3_ref[...])
      bn_s[0:1, :] = sc1
      bn_s[1:2, :] = sh1_
      bn_s[2:3, :] = sc2
      bn_s[3:4, :] = sh2_
      bn3_s[0:1, :] = jnp.concatenate([sc3] * _P, axis=1)
      bn3_s[1:2, :] = jnp.concatenate([sh3_] * _P, axis=1)

    h1 = jnp.maximum(
        jnp.dot(_xcp(xr_ref, c_ref), wd1_ref[...],
                precision=_DEFAULT) * bn_s[0:1, :] + bn_s[1:2, :], 0.0)
    h2 = jnp.maximum(
        jnp.dot(h1, wd2_ref[...],
                precision=_DEFAULT) * bn_s[2:3, :] + bn_s[3:4, :], 0.0)
    h3 = jnp.maximum(
        jnp.dot(h2, wd3_ref[...],
                precision=_DEFAULT) * bn3_s[0:1, :] + bn3_s[1:2, :], 0.0)
    m = h3[:, 0:64]
    for d in range(1, _P):
      m = jnp.maximum(m, h3[:, d * 64:(d + 1) * 64])
    mr = m.reshape(_MTC, _K // _P, 64)
    ne = (cnt_ref[...] > 0).astype(jnp.float32)
    o_ref[...] = jnp.max(mr, axis=1) * ne

  return pl.pallas_call(
      body, grid=(_NCH,),
      in_specs=[_XRP_SPEC, _CP_SPEC, _full((_PW, _PW)), _full((_W, 32)),
                _full((1, 32)), _full((1, 32)), _full((_PW, _PW)),
                _full((32, 32)), _full((1, 32)), _full((1, 32)),
                _full((_PW, _P * 64)), _full((32, 64)), _full((1, 64)),
                _full((1, 64)), _full((_PW, _PW)), _full((1, _PW)),
                _full((_PW, _PW)), _full((1, _PW)),
                _full((_PW, _PW)), _full((1, _PW)),
                pl.BlockSpec((_MTC, 1), lambda i: (i, 0))],
      out_specs=pl.BlockSpec((_MTC, 64), lambda i: (i, 0)),
      out_shape=jax.ShapeDtypeStruct((_B * _M, 64), jnp.float32),
      scratch_shapes=[pltpu.VMEM((4, _PW), jnp.float32),
                      pltpu.VMEM((2, pw3), jnp.float32)],
  )(xrp, cp, wd1, w1, g1, b1, wd2, w2, g2, b2, wd3, w3, g3, b3,
    gx, sx, gh1, sh1, gh2, sh2, cnt2)


def kernel(xyz, new_xyz, feats, W1, g1, b1, W2, g2, b2, W3, g3, b3):
  xyzf = jnp.transpose(xyz, (0, 2, 1)).reshape(-1)
  newf = jnp.transpose(new_xyz, (0, 2, 1)).reshape(-1)
  idxg, cnt = _make_ball_query()(xyzf, newf)

  ci = feats.shape[-1]
  tab = jnp.concatenate(
      [xyz, feats, jnp.zeros((_B, _N, _W - 3 - ci), jnp.float32)],
      axis=-1).reshape(_B * _N, _W)
  xr = _make_gather()(tab, idxg)

  cpad = jnp.concatenate(
      [new_xyz.reshape(_B * _M, 3),
       jnp.zeros((_B * _M, _W - 3), jnp.float32)], axis=1)
  cp = jnp.tile(cpad, (1, _P))
  w1p = jnp.concatenate(
      [W1, jnp.zeros((_W - W1.shape[0], W1.shape[1]), jnp.float32)], axis=0)
  eye = jnp.eye(_P, dtype=jnp.float32)
  wd1 = jnp.kron(eye, w1p)
  wd2 = jnp.kron(eye, W2)
  wd3 = jnp.kron(eye, W3)
  g1r, b1r = g1.reshape(1, -1), b1.reshape(1, -1)
  g2r, b2r = g2.reshape(1, -1), b2.reshape(1, -1)
  g3r, b3r = g3.reshape(1, -1), b3.reshape(1, -1)

  xrp = xr.reshape(_PROWS, _PW)
  gx, sx = _p1(xrp, cp)
  gh1, sh1 = _p2(xrp, cp, wd1, w1p, g1r, b1r, gx, sx)
  gh2, sh2 = _p3(xrp, cp, wd1, w1p, g1r, b1r, wd2, W2, g2r, b2r,
                 gx, sx, gh1, sh1)
  out = _p4(xrp, cp, wd1, w1p, g1r, b1r, wd2, W2, g2r, b2r, wd3, W3, g3r,
            b3r, gx, sx, gh1, sh1, gh2, sh2, cnt.reshape(_B * _M, 1))
  return out.reshape(_B, _M, 64)


# speedup vs baseline: 1.0724x; 1.0724x over previous
import functools

import jax
import jax.numpy as jnp
from jax import lax
from jax.experimental import pallas as pl
from jax.experimental.pallas import tpu as pltpu
from jax.experimental.pallas import tpu_sc as plsc

_RADIUS = 0.2
_K = 32
_EPS = 1e-5

_B, _N, _M = 2, 8192, 2048
_W = 32
_NW = 32
_MPW = _M // _NW
_ROWS = _B * _M * _K
_RT = 8192
_GRID = _ROWS // _RT
_HIGHEST = lax.Precision.HIGHEST
_DEFAULT = lax.Precision.DEFAULT


def _make_ball_query():
  mesh = plsc.VectorSubcoreMesh(core_axis_name="c", subcore_axis_name="s")

  gch = 128
  ngch = _MPW * _K // gch

  @functools.partial(
      pl.kernel,
      mesh=mesh,
      compiler_params=pltpu.CompilerParams(
          needs_layout_passes=False, use_tc_tiling_on_sc=False),
      out_type=(
          jax.ShapeDtypeStruct((_ROWS, _W), jnp.float32),
          jax.ShapeDtypeStruct((_B * _M,), jnp.int32),
      ),
      scratch_types=[
          pltpu.VMEM((_N,), jnp.float32),
          pltpu.VMEM((_N,), jnp.float32),
          pltpu.VMEM((_N,), jnp.float32),
          pltpu.VMEM((_MPW,), jnp.float32),
          pltpu.VMEM((_MPW,), jnp.float32),
          pltpu.VMEM((_MPW,), jnp.float32),
          pltpu.VMEM((128,), jnp.int32),
          pltpu.VMEM((_MPW * _K,), jnp.int32),
          pltpu.VMEM((_MPW,), jnp.int32),
          pltpu.VMEM((2, 128, _W), jnp.float32),
          pltpu.SemaphoreType.DMA,
      ],
  )
  def ball_query_kernel(xyzf_hbm, newf_hbm, tab_hbm, xr_hbm, cnt_hbm,
                        x_v, y_v, z_v, cx_v, cy_v, cz_v,
                        list_v, oidx_v, ocnt_v, rows_v, gsem):
    wid = lax.axis_index("s") * 2 + lax.axis_index("c")
    iota = lax.iota(jnp.int32, 16)
    zeros16 = jnp.zeros((16,), jnp.int32)
    r2 = jnp.float32(_RADIUS * _RADIUS)
    for b in range(_B):
      pltpu.sync_copy(xyzf_hbm.at[pl.ds((b * 3 + 0) * _N, _N)], x_v)
      pltpu.sync_copy(xyzf_hbm.at[pl.ds((b * 3 + 1) * _N, _N)], y_v)
      pltpu.sync_copy(xyzf_hbm.at[pl.ds((b * 3 + 2) * _N, _N)], z_v)
      m0 = wid * _MPW
      pltpu.sync_copy(newf_hbm.at[pl.ds((b * 3 + 0) * _M + m0, _MPW)], cx_v)
      pltpu.sync_copy(newf_hbm.at[pl.ds((b * 3 + 1) * _M + m0, _MPW)], cy_v)
      pltpu.sync_copy(newf_hbm.at[pl.ds((b * 3 + 2) * _M + m0, _MPW)], cz_v)

      def per_centroid(i, carry):
        list_v[pl.ds(0, 16)] = zeros16
        ci = lax.broadcast(i, (16,))
        cx = plsc.load_gather(cx_v, [ci])
        cy = plsc.load_gather(cy_v, [ci])
        cz = plsc.load_gather(cz_v, [ci])

        def cond(c):
          n, cs, _ = c
          return jnp.logical_and(n < _N, cs < _K)

        def body(c):
          n, cs, cv = c
          for u in range(4):
            nv = lax.broadcast(n + u * 16, (16,)) + iota
            xv = plsc.load_gather(x_v, [nv])
            yv = plsc.load_gather(y_v, [nv])
            zv = plsc.load_gather(z_v, [nv])
            dx = xv - cx
            dy = yv - cy
            dz = zv - cz
            d2 = dx * dx + dy * dy + dz * dz
            msk = d2 < r2
            pos = cv + plsc.cumsum(msk.astype(jnp.int32)) - 1
            plsc.store_scatter(list_v, [pos], nv, mask=msk)
            cv = cv + plsc.all_reduce_population_count(msk)
          return (n + 64, jnp.max(cv), cv)

        _, cs, _ = lax.while_loop(
            cond, body, (jnp.int32(0), jnp.int32(0), zeros16))
        cnt = jnp.minimum(cs, _K)
        cntv = lax.broadcast(cnt, (16,))
        first = plsc.load_gather(list_v, [zeros16])
        for h in range(2):
          slots = iota + (h * 16)
          vals = list_v[pl.ds(h * 16, 16)]
          vals = jnp.where(slots < cntv, vals, first) + (b * _N)
          plsc.store_scatter(
              oidx_v, [lax.broadcast(i * _K + h * 16, (16,)) + iota], vals)
        plsc.store_scatter(ocnt_v, [ci], cntv, mask=(iota == 0))
        return carry

      lax.fori_loop(0, _MPW, per_centroid, 0)
      g0 = b * _M + wid * _MPW
      pltpu.sync_copy(ocnt_v, cnt_hbm.at[pl.ds(g0, _MPW)])
      copies = []
      for c in range(ngch):
        copies.append(pltpu.async_copy(
            tab_hbm.at[oidx_v.at[pl.ds(c * gch, gch)]],
            rows_v.at[c % 2], gsem))
        if c > 0:
          copies[c - 1].wait()
          pltpu.sync_copy(
              rows_v.at[(c - 1) % 2],
              xr_hbm.at[pl.ds(g0 * _K + (c - 1) * gch, gch)])
      copies[ngch - 1].wait()
      pltpu.sync_copy(
          rows_v.at[(ngch - 1) % 2],
          xr_hbm.at[pl.ds(g0 * _K + (ngch - 1) * gch, gch)])

  return ball_query_kernel


_P = 8
_PW = _P * _W
_PROWS = _ROWS // _P
_CHP = 4096
_NCH = _PROWS // _CHP
_MTC = _CHP * _P // _K


def _true_stats(gp, sp, width=_W):
  g = jnp.zeros((width, width), jnp.float32)
  s = jnp.zeros((1, width), jnp.float32)
  for d in range(_P):
    g = g + lax.slice(gp, (d * width, d * width),
                      ((d + 1) * width, (d + 1) * width))
    s = s + lax.slice(sp, (0, d * width), (1, (d + 1) * width))
  return g, s


def _bn_affine(g_mat, s_vec, w_mat, gamma, beta):
  mean = jnp.dot(s_vec, w_mat, precision=_HIGHEST) / _ROWS
  m2 = jnp.sum(w_mat * jnp.dot(g_mat, w_mat, precision=_HIGHEST),
               axis=0, keepdims=True) / _ROWS
  var = m2 - mean * mean
  scale = gamma * lax.rsqrt(var + _EPS)
  shift = beta - mean * scale
  return scale, shift


def _bn_packed(gp_ref, sp_ref, w_ref, gamma_ref, beta_ref, wpad=None):
  g, s = _true_stats(gp_ref[...], sp_ref[...])
  w = w_ref[...] if wpad is None else w_ref[...][:_W - wpad, :]
  sc, sh = _bn_affine(g, s, w, gamma_ref[...], beta_ref[...])
  return (jnp.concatenate([sc] * _P, axis=1),
          jnp.concatenate([sh] * _P, axis=1))


def _xcp(xr_ref, c_ref):
  x = xr_ref[...]
  cw = c_ref[...]
  cexp = jnp.broadcast_to(
      cw[:, None, :], (_MTC, _K // _P, _PW)).reshape(_CHP, _PW)
  return x - cexp


def _accum(i, g_ref, s_ref, h):
  @pl.when(i == 0)
  def _():
    g_ref[...] = jnp.zeros_like(g_ref)
    s_ref[...] = jnp.zeros_like(s_ref)
  g_ref[...] += lax.dot_general(h, h, (((0,), (0,)), ((), ())),
                                precision=_DEFAULT)
  s_ref[...] += jnp.sum(h, axis=0, keepdims=True)


def _full(shape):
  return pl.BlockSpec(shape, lambda i: tuple(0 for _ in shape))


_XRP_SPEC = pl.BlockSpec((_CHP, _PW), lambda i: (i, 0))
_CP_SPEC = pl.BlockSpec((_MTC, _PW), lambda i: (i, 0))
_GP_OUT = [_full((_PW, _PW)), _full((1, _PW))]
_GP_SHAPE = [jax.ShapeDtypeStruct((_PW, _PW), jnp.float32),
             jax.ShapeDtypeStruct((1, _PW), jnp.float32)]


def _p1(xrp, cp):
  def body(xr_ref, c_ref, g_ref, s_ref):
    _accum(pl.program_id(0), g_ref, s_ref, _xcp(xr_ref, c_ref))

  return pl.pallas_call(
      body, grid=(_NCH,),
      in_specs=[_XRP_SPEC, _CP_SPEC],
      out_specs=_GP_OUT, out_shape=_GP_SHAPE,
  )(xrp, cp)


def _p2(xrp, cp, wd1, w1, g1, b1, gx, sx):
  def body(xr_ref, c_ref, wd1_ref, w1_ref, g1_ref, b1_ref, gx_ref, sx_ref,
           g_ref, s_ref, bn_s):
    i = pl.program_id(0)

    @pl.when(i == 0)
    def _():
      sc, sh = _bn_packed(gx_ref, sx_ref, w1_ref, g1_ref, b1_ref)
      bn_s[0:1, :] = sc
      bn_s[1:2, :] = sh

    h1 = jnp.maximum(
        jnp.dot(_xcp(xr_ref, c_ref), wd1_ref[...],
                precision=_DEFAULT) * bn_s[0:1, :] + bn_s[1:2, :], 0.0)
    _accum(i, g_ref, s_ref, h1)

  return pl.pallas_call(
      body, grid=(_NCH,),
      in_specs=[_XRP_SPEC, _CP_SPEC, _full((_PW, _PW)), _full((_W, 32)),
                _full((1, 32)), _full((1, 32)), _full((_PW, _PW)),
                _full((1, _PW))],
      out_specs=_GP_OUT, out_shape=_GP_SHAPE,
      scratch_shapes=[pltpu.VMEM((2, _PW), jnp.float32)],
  )(xrp, cp, wd1, w1, g1, b1, gx, sx)


def _p3(xrp, cp, wd1, w1, g1, b1, wd2, w2, g2, b2, gx, sx, gh1, sh1):
  def body(xr_ref, c_ref, wd1_ref, w1_ref, g1_ref, b1_ref,
           wd2_ref, w2_ref, g2_ref, b2_ref,
           gx_ref, sx_ref, gh1_ref, sh1_ref, g_ref, s_ref, bn_s):
    i = pl.program_id(0)

    @pl.when(i == 0)
    def _():
      sc1, sh1_ = _bn_packed(gx_ref, sx_ref, w1_ref, g1_ref, b1_ref)
      sc2, sh2_ = _bn_packed(gh1_ref, sh1_ref, w2_ref, g2_ref, b2_ref)
      bn_s[0:1, :] = sc1
      bn_s[1:2, :] = sh1_
      bn_s[2:3, :] = sc2
      bn_s[3:4, :] = sh2_

    h1 = jnp.maximum(
        jnp.dot(_xcp(xr_ref, c_ref), wd1_ref[...],
                precision=_DEFAULT) * bn_s[0:1, :] + bn_s[1:2, :], 0.0)
    h2 = jnp.maximum(
        jnp.dot(h1, wd2_ref[...],
                precision=_DEFAULT) * bn_s[2:3, :] + bn_s[3:4, :], 0.0)
    _accum(i, g_ref, s_ref, h2)

  return pl.pallas_call(
      body, grid=(_NCH,),
      in_specs=[_XRP_SPEC, _CP_SPEC, _full((_PW, _PW)), _full((_W, 32)),
                _full((1, 32)), _full((1, 32)), _full((_PW, _PW)),
                _full((32, 32)), _full((1, 32)), _full((1, 32)),
                _full((_PW, _PW)), _full((1, _PW)),
                _full((_PW, _PW)), _full((1, _PW))],
      out_specs=_GP_OUT, out_shape=_GP_SHAPE,
      scratch_shapes=[pltpu.VMEM((4, _PW), jnp.float32)],
  )(xrp, cp, wd1, w1, g1, b1, wd2, w2, g2, b2, gx, sx, gh1, sh1)


def _p4(xrp, cp, wd1, w1, g1, b1, wd2, w2, g2, b2, wd3, w3, g3, b3,
        gx, sx, gh1, sh1, gh2, sh2, cnt2):
  pw3 = _P * 64

  def body(xr_ref, c_ref, wd1_ref, w1_ref, g1_ref, b1_ref,
           wd2_ref, w2_ref, g2_ref, b2_ref, wd3_ref, w3_ref, g3_ref, b3_ref,
           gx_ref, sx_ref, gh1_ref, sh1_ref, gh2_ref, sh2_ref,
           cnt_ref, o_ref, bn_s, bn3_s):
    i = pl.program_id(0)

    @pl.when(i == 0)
    def _():
      sc1, sh1_ = _bn_packed(gx_ref, sx_ref, w1_ref, g1_ref, b1_ref)
      sc2, sh2_ = _bn_packed(gh1_ref, sh1_ref, w2_ref, g2_ref, b2_ref)
      g3t, s3t = _true_stats(gh2_ref[...], sh2_ref[...])
      sc3, sh3_ = _bn_affine(g3t, s3t, w3_ref[...], g3_ref[...], b3_ref[...])
      bn_s[0:1, :] = sc1
      bn_s[1:2, :] = sh1_
      bn_s[2:3, :] = sc2
      bn_s[3:4, :] = sh2_
      bn3_s[0:1, :] = jnp.concatenate([sc3] * _P, axis=1)
      bn3_s[1:2, :] = jnp.concatenate([sh3_] * _P, axis=1)

    h1 = jnp.maximum(
        jnp.dot(_xcp(xr_ref, c_ref), wd1_ref[...],
                precision=_DEFAULT) * bn_s[0:1, :] + bn_s[1:2, :], 0.0)
    h2 = jnp.maximum(
        jnp.dot(h1, wd2_ref[...],
                precision=_DEFAULT) * bn_s[2:3, :] + bn_s[3:4, :], 0.0)
    h3 = jnp.maximum(
        jnp.dot(h2, wd3_ref[...],
                precision=_DEFAULT) * bn3_s[0:1, :] + bn3_s[1:2, :], 0.0)
    m = h3[:, 0:64]
    for d in range(1, _P):
      m = jnp.maximum(m, h3[:, d * 64:(d + 1) * 64])
    mr = m.reshape(_MTC, _K // _P, 64)
    ne = (cnt_ref[...] > 0).astype(jnp.float32)
    o_ref[...] = jnp.max(mr, axis=1) * ne

  return pl.pallas_call(
      body, grid=(_NCH,),
      in_specs=[_XRP_SPEC, _CP_SPEC, _full((_PW, _PW)), _full((_W, 32)),
                _full((1, 32)), _full((1, 32)), _full((_PW, _PW)),
                _full((32, 32)), _full((1, 32)), _full((1, 32)),
                _full((_PW, _P * 64)), _full((32, 64)), _full((1, 64)),
                _full((1, 64)), _full((_PW, _PW)), _full((1, _PW)),
                _full((_PW, _PW)), _full((1, _PW)),
                _full((_PW, _PW)), _full((1, _PW)),
                pl.BlockSpec((_MTC, 1), lambda i: (i, 0))],
      out_specs=pl.BlockSpec((_MTC, 64), lambda i: (i, 0)),
      out_shape=jax.ShapeDtypeStruct((_B * _M, 64), jnp.float32),
      scratch_shapes=[pltpu.VMEM((4, _PW), jnp.float32),
                      pltpu.VMEM((2, pw3), jnp.float32)],
  )(xrp, cp, wd1, w1, g1, b1, wd2, w2, g2, b2, wd3, w3, g3, b3,
    gx, sx, gh1, sh1, gh2, sh2, cnt2)


def kernel(xyz, new_xyz, feats, W1, g1, b1, W2, g2, b2, W3, g3, b3):
  xyzf = jnp.transpose(xyz, (0, 2, 1)).reshape(-1)
  newf = jnp.transpose(new_xyz, (0, 2, 1)).reshape(-1)
  ci = feats.shape[-1]
  tab = jnp.concatenate(
      [xyz, feats, jnp.zeros((_B, _N, _W - 3 - ci), jnp.float32)],
      axis=-1).reshape(_B * _N, _W)
  xr, cnt = _make_ball_query()(xyzf, newf, tab)

  cpad = jnp.concatenate(
      [new_xyz.reshape(_B * _M, 3),
       jnp.zeros((_B * _M, _W - 3), jnp.float32)], axis=1)
  cp = jnp.tile(cpad, (1, _P))
  w1p = jnp.concatenate(
      [W1, jnp.zeros((_W - W1.shape[0], W1.shape[1]), jnp.float32)], axis=0)
  eye = jnp.eye(_P, dtype=jnp.float32)
  wd1 = jnp.kron(eye, w1p)
  wd2 = jnp.kron(eye, W2)
  wd3 = jnp.kron(eye, W3)
  g1r, b1r = g1.reshape(1, -1), b1.reshape(1, -1)
  g2r, b2r = g2.reshape(1, -1), b2.reshape(1, -1)
  g3r, b3r = g3.reshape(1, -1), b3.reshape(1, -1)

  xrp = xr.reshape(_PROWS, _PW)
  gx, sx = _p1(xrp, cp)
  gh1, sh1 = _p2(xrp, cp, wd1, w1p, g1r, b1r, gx, sx)
  gh2, sh2 = _p3(xrp, cp, wd1, w1p, g1r, b1r, wd2, W2, g2r, b2r,
                 gx, sx, gh1, sh1)
  out = _p4(xrp, cp, wd1, w1p, g1r, b1r, wd2, W2, g2r, b2r, wd3, W3, g3r,
            b3r, gx, sx, gh1, sh1, gh2, sh2, cnt.reshape(_B * _M, 1))
  return out.reshape(_B, _M, 64)


# speedup vs baseline: 1.1219x; 1.0462x over previous
import functools

import jax
import jax.numpy as jnp
from jax import lax
from jax.experimental import pallas as pl
from jax.experimental.pallas import tpu as pltpu
from jax.experimental.pallas import tpu_sc as plsc

_RADIUS = 0.2
_K = 32
_EPS = 1e-5

_B, _N, _M = 2, 8192, 2048
_W = 32
_NW = 32
_MPW = _M // _NW
_ROWS = _B * _M * _K
_RT = 8192
_GRID = _ROWS // _RT
_HIGHEST = lax.Precision.HIGHEST
_DEFAULT = lax.Precision.DEFAULT


def _make_ball_query():
  mesh = plsc.VectorSubcoreMesh(core_axis_name="c", subcore_axis_name="s")

  gch = 128
  ngch = _MPW * _K // gch

  @functools.partial(
      pl.kernel,
      mesh=mesh,
      compiler_params=pltpu.CompilerParams(
          needs_layout_passes=False, use_tc_tiling_on_sc=False),
      out_type=(
          jax.ShapeDtypeStruct((_ROWS, _W), jnp.float32),
          jax.ShapeDtypeStruct((_B * _M,), jnp.int32),
      ),
      scratch_types=[
          pltpu.VMEM((_N,), jnp.float32),
          pltpu.VMEM((_N,), jnp.float32),
          pltpu.VMEM((_N,), jnp.float32),
          pltpu.VMEM((_MPW,), jnp.float32),
          pltpu.VMEM((_MPW,), jnp.float32),
          pltpu.VMEM((_MPW,), jnp.float32),
          pltpu.VMEM((192,), jnp.int32),
          pltpu.VMEM((_MPW * _K,), jnp.int32),
          pltpu.VMEM((_MPW,), jnp.int32),
          pltpu.VMEM((2, 128, _W), jnp.float32),
          pltpu.SemaphoreType.DMA,
      ],
  )
  def ball_query_kernel(xyzf_hbm, newf_hbm, tab_hbm, xr_hbm, cnt_hbm,
                        x_v, y_v, z_v, cx_v, cy_v, cz_v,
                        list_v, oidx_v, ocnt_v, rows_v, gsem):
    wid = lax.axis_index("s") * 2 + lax.axis_index("c")
    iota = lax.iota(jnp.int32, 16)
    zeros16 = jnp.zeros((16,), jnp.int32)
    r2 = jnp.float32(_RADIUS * _RADIUS)
    for b in range(_B):
      pltpu.sync_copy(xyzf_hbm.at[pl.ds((b * 3 + 0) * _N, _N)], x_v)
      pltpu.sync_copy(xyzf_hbm.at[pl.ds((b * 3 + 1) * _N, _N)], y_v)
      pltpu.sync_copy(xyzf_hbm.at[pl.ds((b * 3 + 2) * _N, _N)], z_v)
      m0 = wid * _MPW
      pltpu.sync_copy(newf_hbm.at[pl.ds((b * 3 + 0) * _M + m0, _MPW)], cx_v)
      pltpu.sync_copy(newf_hbm.at[pl.ds((b * 3 + 1) * _M + m0, _MPW)], cy_v)
      pltpu.sync_copy(newf_hbm.at[pl.ds((b * 3 + 2) * _M + m0, _MPW)], cz_v)

      def per_centroid(i, carry):
        list_v[pl.ds(0, 16)] = zeros16
        ci = lax.broadcast(i, (16,))
        cx = plsc.load_gather(cx_v, [ci])
        cy = plsc.load_gather(cy_v, [ci])
        cz = plsc.load_gather(cz_v, [ci])

        def cond(c):
          n, cs, _ = c
          return jnp.logical_and(n < _N, cs < _K)

        def body(c):
          n, cs, cv = c
          for u in range(8):
            nv = lax.broadcast(n + u * 16, (16,)) + iota
            xv = plsc.load_gather(x_v, [nv])
            yv = plsc.load_gather(y_v, [nv])
            zv = plsc.load_gather(z_v, [nv])
            dx = xv - cx
            dy = yv - cy
            dz = zv - cz
            d2 = dx * dx + dy * dy + dz * dz
            msk = d2 < r2
            pos = cv + plsc.cumsum(msk.astype(jnp.int32)) - 1
            plsc.store_scatter(list_v, [pos], nv, mask=msk)
            cv = cv + plsc.all_reduce_population_count(msk)
          return (n + 128, jnp.max(cv), cv)

        _, cs, _ = lax.while_loop(
            cond, body, (jnp.int32(0), jnp.int32(0), zeros16))
        cnt = jnp.minimum(cs, _K)
        cntv = lax.broadcast(cnt, (16,))
        first = plsc.load_gather(list_v, [zeros16])
        for h in range(2):
          slots = iota + (h * 16)
          vals = list_v[pl.ds(h * 16, 16)]
          vals = jnp.where(slots < cntv, vals, first) + (b * _N)
          plsc.store_scatter(
              oidx_v, [lax.broadcast(i * _K + h * 16, (16,)) + iota], vals)
        plsc.store_scatter(ocnt_v, [ci], cntv, mask=(iota == 0))
        return carry

      lax.fori_loop(0, _MPW, per_centroid, 0)
      g0 = b * _M + wid * _MPW
      pltpu.sync_copy(ocnt_v, cnt_hbm.at[pl.ds(g0, _MPW)])
      copies = []
      for c in range(ngch):
        copies.append(pltpu.async_copy(
            tab_hbm.at[oidx_v.at[pl.ds(c * gch, gch)]],
            rows_v.at[c % 2], gsem))
        if c > 0:
          copies[c - 1].wait()
          pltpu.sync_copy(
              rows_v.at[(c - 1) % 2],
              xr_hbm.at[pl.ds(g0 * _K + (c - 1) * gch, gch)])
      copies[ngch - 1].wait()
      pltpu.sync_copy(
          rows_v.at[(ngch - 1) % 2],
          xr_hbm.at[pl.ds(g0 * _K + (ngch - 1) * gch, gch)])

  return ball_query_kernel


_P = 8
_PW = _P * _W
_PROWS = _ROWS // _P
_CHP = 4096
_NCH = _PROWS // _CHP
_MTC = _CHP * _P // _K


def _true_stats(gp, sp, width=_W):
  g = jnp.zeros((width, width), jnp.float32)
  s = jnp.zeros((1, width), jnp.float32)
  for d in range(_P):
    g = g + lax.slice(gp, (d * width, d * width),
                      ((d + 1) * width, (d + 1) * width))
    s = s + lax.slice(sp, (0, d * width), (1, (d + 1) * width))
  return g, s


def _bn_affine(g_mat, s_vec, w_mat, gamma, beta):
  mean = jnp.dot(s_vec, w_mat, precision=_HIGHEST) / _ROWS
  m2 = jnp.sum(w_mat * jnp.dot(g_mat, w_mat, precision=_HIGHEST),
               axis=0, keepdims=True) / _ROWS
  var = m2 - mean * mean
  scale = gamma * lax.rsqrt(var + _EPS)
  shift = beta - mean * scale
  return scale, shift


def _bn_packed(gp_ref, sp_ref, w_ref, gamma_ref, beta_ref, wpad=None):
  g, s = _true_stats(gp_ref[...], sp_ref[...])
  w = w_ref[...] if wpad is None else w_ref[...][:_W - wpad, :]
  sc, sh = _bn_affine(g, s, w, gamma_ref[...], beta_ref[...])
  return (jnp.concatenate([sc] * _P, axis=1),
          jnp.concatenate([sh] * _P, axis=1))


def _xcp(xr_ref, c_ref):
  x = xr_ref[...]
  cw = c_ref[...]
  cexp = jnp.broadcast_to(
      cw[:, None, :], (_MTC, _K // _P, _PW)).reshape(_CHP, _PW)
  return x - cexp


def _accum(i, g_ref, s_ref, h):
  @pl.when(i == 0)
  def _():
    g_ref[...] = jnp.zeros_like(g_ref)
    s_ref[...] = jnp.zeros_like(s_ref)
  g_ref[...] += lax.dot_general(h, h, (((0,), (0,)), ((), ())),
                                precision=_DEFAULT)
  s_ref[...] += jnp.sum(h, axis=0, keepdims=True)


def _full(shape):
  return pl.BlockSpec(shape, lambda i: tuple(0 for _ in shape))


_XRP_SPEC = pl.BlockSpec((_CHP, _PW), lambda i: (i, 0))
_CP_SPEC = pl.BlockSpec((_MTC, _PW), lambda i: (i, 0))
_GP_OUT = [_full((_PW, _PW)), _full((1, _PW))]
_GP_SHAPE = [jax.ShapeDtypeStruct((_PW, _PW), jnp.float32),
             jax.ShapeDtypeStruct((1, _PW), jnp.float32)]


def _p1(xrp, cp):
  def body(xr_ref, c_ref, g_ref, s_ref):
    _accum(pl.program_id(0), g_ref, s_ref, _xcp(xr_ref, c_ref))

  return pl.pallas_call(
      body, grid=(_NCH,),
      in_specs=[_XRP_SPEC, _CP_SPEC],
      out_specs=_GP_OUT, out_shape=_GP_SHAPE,
  )(xrp, cp)


def _p2(xrp, cp, wd1, w1, g1, b1, gx, sx):
  def body(xr_ref, c_ref, wd1_ref, w1_ref, g1_ref, b1_ref, gx_ref, sx_ref,
           g_ref, s_ref, bn_s):
    i = pl.program_id(0)

    @pl.when(i == 0)
    def _():
      sc, sh = _bn_packed(gx_ref, sx_ref, w1_ref, g1_ref, b1_ref)
      bn_s[0:1, :] = sc
      bn_s[1:2, :] = sh

    h1 = jnp.maximum(
        jnp.dot(_xcp(xr_ref, c_ref), wd1_ref[...],
                precision=_DEFAULT) * bn_s[0:1, :] + bn_s[1:2, :], 0.0)
    _accum(i, g_ref, s_ref, h1)

  return pl.pallas_call(
      body, grid=(_NCH,),
      in_specs=[_XRP_SPEC, _CP_SPEC, _full((_PW, _PW)), _full((_W, 32)),
                _full((1, 32)), _full((1, 32)), _full((_PW, _PW)),
                _full((1, _PW))],
      out_specs=_GP_OUT, out_shape=_GP_SHAPE,
      scratch_shapes=[pltpu.VMEM((2, _PW), jnp.float32)],
  )(xrp, cp, wd1, w1, g1, b1, gx, sx)


def _p3(xrp, cp, wd1, w1, g1, b1, wd2, w2, g2, b2, gx, sx, gh1, sh1):
  def body(xr_ref, c_ref, wd1_ref, w1_ref, g1_ref, b1_ref,
           wd2_ref, w2_ref, g2_ref, b2_ref,
           gx_ref, sx_ref, gh1_ref, sh1_ref, g_ref, s_ref, bn_s):
    i = pl.program_id(0)

    @pl.when(i == 0)
    def _():
      sc1, sh1_ = _bn_packed(gx_ref, sx_ref, w1_ref, g1_ref, b1_ref)
      sc2, sh2_ = _bn_packed(gh1_ref, sh1_ref, w2_ref, g2_ref, b2_ref)
      bn_s[0:1, :] = sc1
      bn_s[1:2, :] = sh1_
      bn_s[2:3, :] = sc2
      bn_s[3:4, :] = sh2_

    h1 = jnp.maximum(
        jnp.dot(_xcp(xr_ref, c_ref), wd1_ref[...],
                precision=_DEFAULT) * bn_s[0:1, :] + bn_s[1:2, :], 0.0)
    h2 = jnp.maximum(
        jnp.dot(h1, wd2_ref[...],
                precision=_DEFAULT) * bn_s[2:3, :] + bn_s[3:4, :], 0.0)
    _accum(i, g_ref, s_ref, h2)

  return pl.pallas_call(
      body, grid=(_NCH,),
      in_specs=[_XRP_SPEC, _CP_SPEC, _full((_PW, _PW)), _full((_W, 32)),
                _full((1, 32)), _full((1, 32)), _full((_PW, _PW)),
                _full((32, 32)), _full((1, 32)), _full((1, 32)),
                _full((_PW, _PW)), _full((1, _PW)),
                _full((_PW, _PW)), _full((1, _PW))],
      out_specs=_GP_OUT, out_shape=_GP_SHAPE,
      scratch_shapes=[pltpu.VMEM((4, _PW), jnp.float32)],
  )(xrp, cp, wd1, w1, g1, b1, wd2, w2, g2, b2, gx, sx, gh1, sh1)


def _p4(xrp, cp, wd1, w1, g1, b1, wd2, w2, g2, b2, wd3, w3, g3, b3,
        gx, sx, gh1, sh1, gh2, sh2, cnt2):
  pw3 = _P * 64

  def body(xr_ref, c_ref, wd1_ref, w1_ref, g1_ref, b1_ref,
           wd2_ref, w2_ref, g2_ref, b2_ref, wd3_ref, w3_ref, g3_ref, b3_ref,
           gx_ref, sx_ref, gh1_ref, sh1_ref, gh2_ref, sh2_ref,
           cnt_ref, o_ref, bn_s, bn3_s):
    i = pl.program_id(0)

    @pl.when(i == 0)
    def _():
      sc1, sh1_ = _bn_packed(gx_ref, sx_ref, w1_ref, g1_ref, b1_ref)
      sc2, sh2_ = _bn_packed(gh1_ref, sh1_ref, w2_ref, g2_ref, b2_ref)
      g3t, s3t = _true_stats(gh2_ref[...], sh2_ref[...])
      sc3, sh3_ = _bn_affine(g3t, s3t, w3_ref[...], g3_ref[...], b3_ref[...])
      bn_s[0:1, :] = sc1
      bn_s[1:2, :] = sh1_
      bn_s[2:3, :] = sc2
      bn_s[3:4, :] = sh2_
      bn3_s[0:1, :] = jnp.concatenate([sc3] * _P, axis=1)
      bn3_s[1:2, :] = jnp.concatenate([sh3_] * _P, axis=1)

    h1 = jnp.maximum(
        jnp.dot(_xcp(xr_ref, c_ref), wd1_ref[...],
                precision=_DEFAULT) * bn_s[0:1, :] + bn_s[1:2, :], 0.0)
    h2 = jnp.maximum(
        jnp.dot(h1, wd2_ref[...],
                precision=_DEFAULT) * bn_s[2:3, :] + bn_s[3:4, :], 0.0)
    h3 = jnp.maximum(
        jnp.dot(h2, wd3_ref[...],
                precision=_DEFAULT) * bn3_s[0:1, :] + bn3_s[1:2, :], 0.0)
    m = h3[:, 0:64]
    for d in range(1, _P):
      m = jnp.maximum(m, h3[:, d * 64:(d + 1) * 64])
    mr = m.reshape(_MTC, _K // _P, 64)
    ne = (cnt_ref[...] > 0).astype(jnp.float32)
    o_ref[...] = jnp.max(mr, axis=1) * ne

  return pl.pallas_call(
      body, grid=(_NCH,),
      in_specs=[_XRP_SPEC, _CP_SPEC, _full((_PW, _PW)), _full((_W, 32)),
                _full((1, 32)), _full((1, 32)), _full((_PW, _PW)),
                _full((32, 32)), _full((1, 32)), _full((1, 32)),
                _full((_PW, _P * 64)), _full((32, 64)), _full((1, 64)),
                _full((1, 64)), _full((_PW, _PW)), _full((1, _PW)),
                _full((_PW, _PW)), _full((1, _PW)),
                _full((_PW, _PW)), _full((1, _PW)),
                pl.BlockSpec((_MTC, 1), lambda i: (i, 0))],
      out_specs=pl.BlockSpec((_MTC, 64), lambda i: (i, 0)),
      out_shape=jax.ShapeDtypeStruct((_B * _M, 64), jnp.float32),
      scratch_shapes=[pltpu.VMEM((4, _PW), jnp.float32),
                      pltpu.VMEM((2, pw3), jnp.float32)],
  )(xrp, cp, wd1, w1, g1, b1, wd2, w2, g2, b2, wd3, w3, g3, b3,
    gx, sx, gh1, sh1, gh2, sh2, cnt2)


def kernel(xyz, new_xyz, feats, W1, g1, b1, W2, g2, b2, W3, g3, b3):
  xyzf = jnp.transpose(xyz, (0, 2, 1)).reshape(-1)
  newf = jnp.transpose(new_xyz, (0, 2, 1)).reshape(-1)
  ci = feats.shape[-1]
  tab = jnp.concatenate(
      [xyz, feats, jnp.zeros((_B, _N, _W - 3 - ci), jnp.float32)],
      axis=-1).reshape(_B * _N, _W)
  xr, cnt = _make_ball_query()(xyzf, newf, tab)

  cpad = jnp.concatenate(
      [new_xyz.reshape(_B * _M, 3),
       jnp.zeros((_B * _M, _W - 3), jnp.float32)], axis=1)
  cp = jnp.tile(cpad, (1, _P))
  w1p = jnp.concatenate(
      [W1, jnp.zeros((_W - W1.shape[0], W1.shape[1]), jnp.float32)], axis=0)
  eye = jnp.eye(_P, dtype=jnp.float32)
  wd1 = jnp.kron(eye, w1p)
  wd2 = jnp.kron(eye, W2)
  wd3 = jnp.kron(eye, W3)
  g1r, b1r = g1.reshape(1, -1), b1.reshape(1, -1)
  g2r, b2r = g2.reshape(1, -1), b2.reshape(1, -1)
  g3r, b3r = g3.reshape(1, -1), b3.reshape(1, -1)

  xrp = xr.reshape(_PROWS, _PW)
  gx, sx = _p1(xrp, cp)
  gh1, sh1 = _p2(xrp, cp, wd1, w1p, g1r, b1r, gx, sx)
  gh2, sh2 = _p3(xrp, cp, wd1, w1p, g1r, b1r, wd2, W2, g2r, b2r,
                 gx, sx, gh1, sh1)
  out = _p4(xrp, cp, wd1, w1p, g1r, b1r, wd2, W2, g2r, b2r, wd3, W3, g3r,
            b3r, gx, sx, gh1, sh1, gh2, sh2, cnt.reshape(_B * _M, 1))
  return out.reshape(_B, _M, 64)
